# baseline (speedup 1.0000x reference)
import functools

import jax
import jax.numpy as jnp
from jax import lax
from jax.experimental import pallas as pl
from jax.experimental.pallas import tpu as pltpu
from jax.experimental.pallas import tpu_sc as plsc

T = 2048
D = 1024
E = 8
F = 512
SF = 512

BLK = 128
P = 2 * T + E * BLK
NBLK = P // BLK
NW = 32
CH = T // NW
SUB = 8
CHK = 256


def _route_kernel(x_ref, gw_ref, p1_ref, p2_ref, w1_ref, w2_ref, be_ref):
    x = x_ref[...]
    logits = jnp.dot(x, gw_ref[...].T, preferred_element_type=jnp.float32)
    logits = logits - jnp.max(logits, axis=-1, keepdims=True)
    ex = jnp.exp(logits)
    probs = ex / jnp.sum(ex, axis=-1, keepdims=True)

    col = lax.broadcasted_iota(jnp.int32, (T, E), 1)
    a1 = jnp.argmax(probs, axis=-1)
    m1 = jnp.max(probs, axis=-1)
    oh1 = (col == a1[:, None]).astype(jnp.float32)
    masked = jnp.where(oh1 > 0, -jnp.inf, probs)
    a2 = jnp.argmax(masked, axis=-1)
    m2 = jnp.max(masked, axis=-1)
    oh2 = (col == a2[:, None]).astype(jnp.float32)
    s = m1 + m2

    M = oh1 + oh2
    ri = lax.broadcasted_iota(jnp.int32, (CHK, CHK), 0)
    ci = lax.broadcasted_iota(jnp.int32, (CHK, CHK), 1)
    tril = (ci < ri).astype(jnp.float32)
    acc = jnp.zeros((1, E), jnp.float32)
    segs = []
    for ch in range(T // CHK):
        Mc = M[ch * CHK:(ch + 1) * CHK]
        segs.append(jnp.dot(tril, Mc, preferred_element_type=jnp.float32) + acc)
        acc = acc + jnp.sum(Mc, axis=0, keepdims=True)
    S = jnp.concatenate(segs, axis=0)
    counts = acc

    padded = jnp.ceil(counts * (1.0 / BLK)) * BLK
    er = lax.broadcasted_iota(jnp.int32, (E, E), 0)
    ec = lax.broadcasted_iota(jnp.int32, (E, E), 1)
    upper = (er < ec).astype(jnp.float32)
    off = jnp.dot(padded, upper, preferred_element_type=jnp.float32)

    rank1 = jnp.sum(S * oh1, axis=1)
    rank2 = jnp.sum(S * oh2, axis=1)
    base1 = jnp.sum(off * oh1, axis=1)
    base2 = jnp.sum(off * oh2, axis=1)
    p1_ref[...] = (base1 + rank1).astype(jnp.int32).reshape(1, T)
    p2_ref[...] = (base2 + rank2).astype(jnp.int32).reshape(1, T)
    w1_ref[...] = jnp.broadcast_to((m1 / s)[:, None], (T, 16))
    w2_ref[...] = jnp.broadcast_to((m2 / s)[:, None], (T, 16))

    ends = off + padded
    ends_b = jnp.broadcast_to(ends, (NBLK, E))
    sb = lax.broadcasted_iota(jnp.int32, (NBLK, E), 0).astype(
        jnp.float32) * float(BLK)
    cnt = jnp.sum((ends_b <= sb).astype(jnp.int32), axis=1)
    be_ref[...] = jnp.minimum(cnt, E - 1).reshape(1, NBLK)


def _route(x, gate_w):
    return pl.pallas_call(
        _route_kernel,
        out_shape=(
            jax.ShapeDtypeStruct((1, T), jnp.int32),
            jax.ShapeDtypeStruct((1, T), jnp.int32),
            jax.ShapeDtypeStruct((T, 16), jnp.float32),
            jax.ShapeDtypeStruct((T, 16), jnp.float32),
            jax.ShapeDtypeStruct((1, NBLK), jnp.int32),
        ),
    )(x, gate_w)


def _shared_kernel(x_ref, sgu_ref, sdown_ref, out_ref):
    sh = jnp.dot(x_ref[...], sgu_ref[...], preferred_element_type=jnp.float32)
    sg = sh[:, :SF]
    su = sh[:, SF:]
    act = (sg * jax.nn.sigmoid(sg)) * su
    out_ref[...] = jnp.dot(act, sdown_ref[...], preferred_element_type=jnp.float32)


def _shared(x, sgu, sdown):
    return pl.pallas_call(
        _shared_kernel,
        out_shape=jax.ShapeDtypeStruct((T, D), jnp.float32),
    )(x, sgu, sdown)


@functools.lru_cache(maxsize=None)
def _make_dispatch():
    mesh = plsc.VectorSubcoreMesh(core_axis_name="c", subcore_axis_name="s")

    @functools.partial(
        pl.kernel,
        mesh=mesh,
        out_type=jax.ShapeDtypeStruct((P, D), jnp.float32),
        scratch_types=[
            pltpu.VMEM((CH,), jnp.int32),
            pltpu.VMEM((CH,), jnp.int32),
            pltpu.VMEM((CH, D), jnp.float32),
            pltpu.SemaphoreType.DMA,
            pltpu.SemaphoreType.DMA,
        ],
    )
    def _dispatch(x_hbm, p1_hbm, p2_hbm, xs_hbm, idx1, idx2, xbuf, sem1, sem2):
        wid = lax.axis_index("s") * 2 + lax.axis_index("c")
        base = wid * CH
        pltpu.sync_copy(p1_hbm.at[pl.ds(base, CH)], idx1)
        pltpu.sync_copy(p2_hbm.at[pl.ds(base, CH)], idx2)
        pltpu.sync_copy(x_hbm.at[pl.ds(base, CH)], xbuf)
        cp1 = pltpu.async_copy(xbuf, xs_hbm.at[idx1], sem1)
        cp2 = pltpu.async_copy(xbuf, xs_hbm.at[idx2], sem2)
        cp1.wait()
        cp2.wait()

    return _dispatch


def _gmm_kernel(be_ref, xs_ref, gu_ref, dn_ref, ys_ref):
    e = be_ref[pl.program_id(0)]
    x = xs_ref[...]

    def mk(k):
        def branch():
            h = jnp.dot(x, gu_ref[k], preferred_element_type=jnp.float32)
            g = h[:, :F]
            u = h[:, F:]
            act = (g * jax.nn.sigmoid(g)) * u
            return jnp.dot(act, dn_ref[k], preferred_element_type=jnp.float32)
        return branch

    ys_ref[...] = lax.switch(e, [mk(k) for k in range(E)])


def _gmm(be, xs, gu, dn):
    grid_spec = pltpu.PrefetchScalarGridSpec(
        num_scalar_prefetch=1,
        grid=(NBLK,),
        in_specs=[
            pl.BlockSpec((BLK, D), lambda b, be: (b, 0)),
            pl.BlockSpec((E, D, 2 * F), lambda b, be: (0, 0, 0)),
            pl.BlockSpec((E, F, D), lambda b, be: (0, 0, 0)),
        ],
        out_specs=pl.BlockSpec((BLK, D), lambda b, be: (b, 0)),
    )
    return pl.pallas_call(
        _gmm_kernel,
        grid_spec=grid_spec,
        out_shape=jax.ShapeDtypeStruct((P, D), jnp.float32),
        compiler_params=pltpu.CompilerParams(
            dimension_semantics=("arbitrary",),
            vmem_limit_bytes=100 * 1024 * 1024,
        ),
    )(be, xs, gu, dn)


@functools.lru_cache(maxsize=None)
def _make_combine():
    mesh = plsc.VectorSubcoreMesh(core_axis_name="c", subcore_axis_name="s")

    nsub = CH // SUB

    @functools.partial(
        pl.kernel,
        mesh=mesh,
        out_type=jax.ShapeDtypeStruct((T, D), jnp.float32),
        scratch_types=[
            pltpu.VMEM((CH,), jnp.int32),
            pltpu.VMEM((CH,), jnp.int32),
            pltpu.VMEM((CH, 16), jnp.float32),
            pltpu.VMEM((2, SUB, D), jnp.float32),
            pltpu.VMEM((2, SUB, D), jnp.float32),
            pltpu.VMEM((2, SUB, D), jnp.float32),
            pltpu.VMEM((2, SUB, D), jnp.float32),
            pltpu.SemaphoreType.DMA,
            pltpu.SemaphoreType.DMA,
            pltpu.SemaphoreType.DMA,
            pltpu.SemaphoreType.DMA,
            pltpu.SemaphoreType.DMA,
            pltpu.SemaphoreType.DMA,
            pltpu.SemaphoreType.DMA,
            pltpu.SemaphoreType.DMA,
        ],
    )
    def _combine(ys_hbm, oi_hbm, p1_hbm, p2_hbm, w1_hbm, out_hbm,
                 idx1, idx2, w1b, y1, y2, oi, ob,
                 g1a, g1b, g2a, g2b, oia, oib, sta, stb):
        wid = lax.axis_index("s") * 2 + lax.axis_index("c")
        base = wid * CH
        gsem = (g1a, g1b, g2a, g2b)
        osem = (oia, oib)
        ssem = (sta, stb)
        pltpu.sync_copy(p1_hbm.at[pl.ds(base, CH)], idx1)
        pltpu.sync_copy(p2_hbm.at[pl.ds(base, CH)], idx2)
        pltpu.sync_copy(w1_hbm.at[pl.ds(base, CH)], w1b)

        pend = {}
        stores = {}

        def start(s):
            slot = s % 2
            rb = base + s * SUB
            pend[s] = (
                pltpu.async_copy(ys_hbm.at[idx1.at[pl.ds(s * SUB, SUB)]],
                                 y1.at[slot], gsem[slot]),
                pltpu.async_copy(ys_hbm.at[idx2.at[pl.ds(s * SUB, SUB)]],
                                 y2.at[slot], gsem[2 + slot]),
                pltpu.async_copy(oi_hbm.at[pl.ds(rb, SUB)], oi.at[slot],
                                 osem[slot]),
            )

        start(0)
        start(1)
        for s in range(nsub):
            slot = s % 2
            rb = base + s * SUB
            for cp in pend.pop(s):
                cp.wait()
            if s >= 2:
                stores.pop(s - 2).wait()

            def row_body(r, carry):
                tok = s * SUB + r
                w1v = w1b[tok, :]
                for cc in range(D // 16):
                    sl = pl.ds(cc * 16, 16)
                    a = y1[slot, r, sl]
                    b = y2[slot, r, sl]
                    ob[slot, r, sl] = oi[slot, r, sl] + a + (1.0 - w1v) * (b - a)
                return carry

            lax.fori_loop(0, SUB, row_body, 0)
            stores[s] = pltpu.async_copy(ob.at[slot],
                                         out_hbm.at[pl.ds(rb, SUB)],
                                         ssem[slot])
            if s + 2 < nsub:
                start(s + 2)
        for s in range(nsub - 2, nsub):
            stores.pop(s).wait()

    return _combine


@jax.jit
def kernel(hidden_states, gate_w, expert_gate_up, expert_down, shared_gate_up,
           shared_down):
    p1w, p2w, w1x, w2x, bew = _route(hidden_states, gate_w)
    p1 = p1w.reshape(T)
    p2 = p2w.reshape(T)
    be = bew.reshape(NBLK)
    out_init = _shared(hidden_states, shared_gate_up, shared_down)
    xs = _make_dispatch()(hidden_states, p1, p2)
    del w2x
    ys = _gmm(be, xs, expert_gate_up, expert_down)
    return _make_combine()(ys, out_init, p1, p2, w1x)


# speedup vs baseline: 1.1529x; 1.1529x over previous
import functools

import jax
import jax.numpy as jnp
from jax import lax
from jax.experimental import pallas as pl
from jax.experimental.pallas import tpu as pltpu
from jax.experimental.pallas import tpu_sc as plsc

T = 2048
D = 1024
E = 8
F = 512
SF = 512

BLK = 128
P = 2 * T + E * BLK
NBLK = P // BLK
NW = 32
CH = T // NW
SUB = 32
CHK = 256


def _route_kernel(x_ref, gw_ref, sgu_ref, sdown_ref, p1_ref, p2_ref, w1_ref,
                  be_ref, oi_ref):
    x = x_ref[...]
    sh = jnp.dot(x, sgu_ref[...], preferred_element_type=jnp.float32)
    sg = sh[:, :SF]
    su = sh[:, SF:]
    sact = (sg * jax.nn.sigmoid(sg)) * su
    oi_ref[...] = jnp.dot(sact, sdown_ref[...], preferred_element_type=jnp.float32)
    logits = jnp.dot(x, gw_ref[...].T, preferred_element_type=jnp.float32)
    logits = logits - jnp.max(logits, axis=-1, keepdims=True)
    ex = jnp.exp(logits)
    probs = ex / jnp.sum(ex, axis=-1, keepdims=True)

    col = lax.broadcasted_iota(jnp.int32, (T, E), 1)
    a1 = jnp.argmax(probs, axis=-1)
    m1 = jnp.max(probs, axis=-1)
    oh1 = (col == a1[:, None]).astype(jnp.float32)
    masked = jnp.where(oh1 > 0, -jnp.inf, probs)
    a2 = jnp.argmax(masked, axis=-1)
    m2 = jnp.max(masked, axis=-1)
    oh2 = (col == a2[:, None]).astype(jnp.float32)
    s = m1 + m2

    M = oh1 + oh2
    ri = lax.broadcasted_iota(jnp.int32, (CHK, CHK), 0)
    ci = lax.broadcasted_iota(jnp.int32, (CHK, CHK), 1)
    tril = (ci < ri).astype(jnp.float32)
    acc = jnp.zeros((1, E), jnp.float32)
    segs = []
    for ch in range(T // CHK):
        Mc = M[ch * CHK:(ch + 1) * CHK]
        segs.append(jnp.dot(tril, Mc, preferred_element_type=jnp.float32) + acc)
        acc = acc + jnp.sum(Mc, axis=0, keepdims=True)
    S = jnp.concatenate(segs, axis=0)
    counts = acc

    padded = jnp.ceil(counts * (1.0 / BLK)) * BLK
    er = lax.broadcasted_iota(jnp.int32, (E, E), 0)
    ec = lax.broadcasted_iota(jnp.int32, (E, E), 1)
    upper = (er < ec).astype(jnp.float32)
    off = jnp.dot(padded, upper, preferred_element_type=jnp.float32)

    rank1 = jnp.sum(S * oh1, axis=1)
    rank2 = jnp.sum(S * oh2, axis=1)
    base1 = jnp.sum(off * oh1, axis=1)
    base2 = jnp.sum(off * oh2, axis=1)
    p1_ref[...] = (base1 + rank1).astype(jnp.int32).reshape(1, T)
    p2_ref[...] = (base2 + rank2).astype(jnp.int32).reshape(1, T)
    w1_ref[...] = jnp.broadcast_to((m1 / s)[:, None], (T, 16))

    ends = off + padded
    ends_b = jnp.broadcast_to(ends, (NBLK, E))
    sb = lax.broadcasted_iota(jnp.int32, (NBLK, E), 0).astype(
        jnp.float32) * float(BLK)
    cnt = jnp.sum((ends_b <= sb).astype(jnp.int32), axis=1)
    be_ref[...] = jnp.minimum(cnt, E - 1).reshape(1, NBLK)


def _route(x, gate_w, sgu, sdown):
    return pl.pallas_call(
        _route_kernel,
        out_shape=(
            jax.ShapeDtypeStruct((1, T), jnp.int32),
            jax.ShapeDtypeStruct((1, T), jnp.int32),
            jax.ShapeDtypeStruct((T, 16), jnp.float32),
            jax.ShapeDtypeStruct((1, NBLK), jnp.int32),
            jax.ShapeDtypeStruct((T, D), jnp.float32),
        ),
        compiler_params=pltpu.CompilerParams(
            vmem_limit_bytes=100 * 1024 * 1024,
        ),
    )(x, gate_w, sgu, sdown)


@functools.lru_cache(maxsize=None)
def _make_dispatch():
    mesh = plsc.VectorSubcoreMesh(core_axis_name="c", subcore_axis_name="s")

    @functools.partial(
        pl.kernel,
        mesh=mesh,
        out_type=jax.ShapeDtypeStruct((P, D), jnp.float32),
        scratch_types=[
            pltpu.VMEM((CH,), jnp.int32),
            pltpu.VMEM((CH,), jnp.int32),
            pltpu.VMEM((CH, D), jnp.float32),
            pltpu.SemaphoreType.DMA,
            pltpu.SemaphoreType.DMA,
        ],
    )
    def _dispatch(x_hbm, p1_hbm, p2_hbm, xs_hbm, idx1, idx2, xbuf, sem1, sem2):
        wid = lax.axis_index("s") * 2 + lax.axis_index("c")
        base = wid * CH
        pltpu.sync_copy(p1_hbm.at[pl.ds(base, CH)], idx1)
        pltpu.sync_copy(p2_hbm.at[pl.ds(base, CH)], idx2)
        pltpu.sync_copy(x_hbm.at[pl.ds(base, CH)], xbuf)
        cp1 = pltpu.async_copy(xbuf, xs_hbm.at[idx1], sem1)
        cp2 = pltpu.async_copy(xbuf, xs_hbm.at[idx2], sem2)
        cp1.wait()
        cp2.wait()

    return _dispatch


def _gmm_kernel(be_ref, xs_ref, gu_ref, dn_ref, ys_ref, gu_cur, dn_cur):
    b = pl.program_id(0)
    e = be_ref[b]
    prev = be_ref[jnp.maximum(b - 1, 0)]

    @pl.when((b == 0) | (e != prev))
    def _load_weights():
        gu_cur[...] = gu_ref[e]
        dn_cur[...] = dn_ref[e]

    h = jnp.dot(xs_ref[...], gu_cur[...], preferred_element_type=jnp.float32)
    g = h[:, :F]
    u = h[:, F:]
    act = (g * jax.nn.sigmoid(g)) * u
    ys_ref[...] = jnp.dot(act, dn_cur[...], preferred_element_type=jnp.float32)


def _gmm(be, xs, gu, dn):
    grid_spec = pltpu.PrefetchScalarGridSpec(
        num_scalar_prefetch=1,
        grid=(NBLK,),
        in_specs=[
            pl.BlockSpec((BLK, D), lambda b, be: (b, 0)),
            pl.BlockSpec((E, D, 2 * F), lambda b, be: (0, 0, 0)),
            pl.BlockSpec((E, F, D), lambda b, be: (0, 0, 0)),
        ],
        out_specs=pl.BlockSpec((BLK, D), lambda b, be: (b, 0)),
        scratch_shapes=[
            pltpu.VMEM((D, 2 * F), jnp.float32),
            pltpu.VMEM((F, D), jnp.float32),
        ],
    )
    return pl.pallas_call(
        _gmm_kernel,
        grid_spec=grid_spec,
        out_shape=jax.ShapeDtypeStruct((P, D), jnp.float32),
        compiler_params=pltpu.CompilerParams(
            dimension_semantics=("arbitrary",),
            vmem_limit_bytes=100 * 1024 * 1024,
        ),
    )(be, xs, gu, dn)


@functools.lru_cache(maxsize=None)
def _make_combine():
    mesh = plsc.VectorSubcoreMesh(core_axis_name="c", subcore_axis_name="s")

    @functools.partial(
        pl.kernel,
        mesh=mesh,
        out_type=jax.ShapeDtypeStruct((T, D), jnp.float32),
        scratch_types=[
            pltpu.VMEM((CH,), jnp.int32),
            pltpu.VMEM((CH,), jnp.int32),
            pltpu.VMEM((CH, 16), jnp.float32),
            pltpu.VMEM((SUB, D), jnp.float32),
            pltpu.VMEM((SUB, D), jnp.float32),
            pltpu.VMEM((SUB, D), jnp.float32),
            pltpu.SemaphoreType.DMA,
            pltpu.SemaphoreType.DMA,
        ],
    )
    def _combine(ys_hbm, oi_hbm, p1_hbm, p2_hbm, w1_hbm, out_hbm,
                 idx1, idx2, w1b, y1, y2, ob, sem1, sem2):
        wid = lax.axis_index("s") * 2 + lax.axis_index("c")
        base = wid * CH
        pltpu.sync_copy(p1_hbm.at[pl.ds(base, CH)], idx1)
        pltpu.sync_copy(p2_hbm.at[pl.ds(base, CH)], idx2)
        pltpu.sync_copy(w1_hbm.at[pl.ds(base, CH)], w1b)
        for sc in range(CH // SUB):
            rb = base + sc * SUB
            cp1 = pltpu.async_copy(ys_hbm.at[idx1.at[pl.ds(sc * SUB, SUB)]], y1,
                                   sem1)
            cp2 = pltpu.async_copy(ys_hbm.at[idx2.at[pl.ds(sc * SUB, SUB)]], y2,
                                   sem2)
            pltpu.sync_copy(oi_hbm.at[pl.ds(rb, SUB)], ob)
            cp1.wait()
            cp2.wait()

            def row_body(r, carry):
                tok = sc * SUB + r
                w1v = w1b[tok, :]
                for cc in range(D // 16):
                    sl = pl.ds(cc * 16, 16)
                    a = y1[r, sl]
                    b = y2[r, sl]
                    ob[r, sl] = ob[r, sl] + a + (1.0 - w1v) * (b - a)
                return carry

            lax.fori_loop(0, SUB, row_body, 0)
            pltpu.sync_copy(ob, out_hbm.at[pl.ds(rb, SUB)])

    return _combine


@jax.jit
def kernel(hidden_states, gate_w, expert_gate_up, expert_down, shared_gate_up,
           shared_down):
    p1w, p2w, w1x, bew, out_init = _route(hidden_states, gate_w,
                                          shared_gate_up, shared_down)
    p1 = p1w.reshape(T)
    p2 = p2w.reshape(T)
    be = bew.reshape(NBLK)
    xs = _make_dispatch()(hidden_states, p1, p2)
    ys = _gmm(be, xs, expert_gate_up, expert_down)
    return _make_combine()(ys, out_init, p1, p2, w1x)


# speedup vs baseline: 1.2888x; 1.1179x over previous
import functools

import jax
import jax.numpy as jnp
from jax import lax
from jax.experimental import pallas as pl
from jax.experimental.pallas import tpu as pltpu
from jax.experimental.pallas import tpu_sc as plsc

T = 2048
D = 1024
E = 8
F = 512
SF = 512

BLK = 128
PBLK = 256
P = 2 * T + E * PBLK
NBLK = P // BLK
NPAIR = P // PBLK
NW = 32
CH = T // NW
SUB = 32
CHK = 256


def _route_kernel(x_ref, gw_ref, p1_ref, p2_ref, w1_ref, be_ref):
    x = x_ref[...]
    logits = jnp.dot(x, gw_ref[...].T, preferred_element_type=jnp.float32)
    logits = logits - jnp.max(logits, axis=-1, keepdims=True)
    ex = jnp.exp(logits)
    probs = ex / jnp.sum(ex, axis=-1, keepdims=True)

    col = lax.broadcasted_iota(jnp.int32, (T, E), 1)
    a1 = jnp.argmax(probs, axis=-1)
    m1 = jnp.max(probs, axis=-1)
    oh1 = (col == a1[:, None]).astype(jnp.float32)
    masked = jnp.where(oh1 > 0, -jnp.inf, probs)
    a2 = jnp.argmax(masked, axis=-1)
    m2 = jnp.max(masked, axis=-1)
    oh2 = (col == a2[:, None]).astype(jnp.float32)
    s = m1 + m2

    M = oh1 + oh2
    ri = lax.broadcasted_iota(jnp.int32, (CHK, CHK), 0)
    ci = lax.broadcasted_iota(jnp.int32, (CHK, CHK), 1)
    tril = (ci < ri).astype(jnp.float32)
    acc = jnp.zeros((1, E), jnp.float32)
    segs = []
    for ch in range(T // CHK):
        Mc = M[ch * CHK:(ch + 1) * CHK]
        segs.append(jnp.dot(tril, Mc, preferred_element_type=jnp.float32) + acc)
        acc = acc + jnp.sum(Mc, axis=0, keepdims=True)
    S = jnp.concatenate(segs, axis=0)
    counts = acc

    padded = jnp.ceil(counts * (1.0 / PBLK)) * PBLK
    er = lax.broadcasted_iota(jnp.int32, (E, E), 0)
    ec = lax.broadcasted_iota(jnp.int32, (E, E), 1)
    upper = (er < ec).astype(jnp.float32)
    off = jnp.dot(padded, upper, preferred_element_type=jnp.float32)

    rank1 = jnp.sum(S * oh1, axis=1)
    rank2 = jnp.sum(S * oh2, axis=1)
    base1 = jnp.sum(off * oh1, axis=1)
    base2 = jnp.sum(off * oh2, axis=1)
    p1_ref[...] = (base1 + rank1).astype(jnp.int32).reshape(1, T)
    p2_ref[...] = (base2 + rank2).astype(jnp.int32).reshape(1, T)
    w1_ref[...] = jnp.broadcast_to((m1 / s)[:, None], (T, 16))

    ends = off + padded
    ends_b = jnp.broadcast_to(ends, (NBLK, E))
    sb = lax.broadcasted_iota(jnp.int32, (NBLK, E), 0).astype(
        jnp.float32) * float(BLK)
    cnt = jnp.sum((ends_b <= sb).astype(jnp.int32), axis=1)
    be_ref[...] = jnp.minimum(cnt, E - 1).reshape(1, NBLK)


def _route(x, gate_w):
    return pl.pallas_call(
        _route_kernel,
        out_shape=(
            jax.ShapeDtypeStruct((1, T), jnp.int32),
            jax.ShapeDtypeStruct((1, T), jnp.int32),
            jax.ShapeDtypeStruct((T, 16), jnp.float32),
            jax.ShapeDtypeStruct((1, NBLK), jnp.int32),
        ),
    )(x, gate_w)


def _shared_kernel(x_ref, sgu_ref, sdown_ref, out_ref):
    sh = jnp.dot(x_ref[...], sgu_ref[...], preferred_element_type=jnp.float32)
    sg = sh[:, :SF]
    su = sh[:, SF:]
    act = (sg * jax.nn.sigmoid(sg)) * su
    out_ref[...] = jnp.dot(act, sdown_ref[...], preferred_element_type=jnp.float32)


def _shared(x, sgu, sdown):
    return pl.pallas_call(
        _shared_kernel,
        out_shape=jax.ShapeDtypeStruct((T, D), jnp.float32),
    )(x, sgu, sdown)


@functools.lru_cache(maxsize=None)
def _make_dispatch():
    mesh = plsc.VectorSubcoreMesh(core_axis_name="c", subcore_axis_name="s")

    @functools.partial(
        pl.kernel,
        mesh=mesh,
        out_type=jax.ShapeDtypeStruct((P, D), jnp.float32),
        scratch_types=[
            pltpu.VMEM((CH,), jnp.int32),
            pltpu.VMEM((CH,), jnp.int32),
            pltpu.VMEM((CH, D), jnp.float32),
            pltpu.SemaphoreType.DMA,
            pltpu.SemaphoreType.DMA,
        ],
    )
    def _dispatch(x_hbm, p1_hbm, p2_hbm, xs_hbm, idx1, idx2, xbuf, sem1, sem2):
        wid = lax.axis_index("s") * 2 + lax.axis_index("c")
        base = wid * CH
        pltpu.sync_copy(p1_hbm.at[pl.ds(base, CH)], idx1)
        pltpu.sync_copy(p2_hbm.at[pl.ds(base, CH)], idx2)
        pltpu.sync_copy(x_hbm.at[pl.ds(base, CH)], xbuf)
        cp1 = pltpu.async_copy(xbuf, xs_hbm.at[idx1], sem1)
        cp2 = pltpu.async_copy(xbuf, xs_hbm.at[idx2], sem2)
        cp1.wait()
        cp2.wait()

    return _dispatch


def _gmm_kernel(be_ref, xs_ref, gu_ref, dn_ref, ys_ref, gu_cur, dn_cur):
    b = pl.program_id(0)
    e = be_ref[2 * b]
    prev = be_ref[jnp.maximum(2 * b - 2, 0)]

    @pl.when((b == 0) | (e != prev))
    def _load_weights():
        gu_cur[...] = gu_ref[e]
        dn_cur[...] = dn_ref[e]

    x0 = xs_ref[:BLK]
    x1 = xs_ref[BLK:]
    h0 = jnp.dot(x0, gu_cur[...], preferred_element_type=jnp.float32)
    h1 = jnp.dot(x1, gu_cur[...], preferred_element_type=jnp.float32)
    a0 = (h0[:, :F] * jax.nn.sigmoid(h0[:, :F])) * h0[:, F:]
    a1 = (h1[:, :F] * jax.nn.sigmoid(h1[:, :F])) * h1[:, F:]
    ys_ref[:BLK] = jnp.dot(a0, dn_cur[...], preferred_element_type=jnp.float32)
    ys_ref[BLK:] = jnp.dot(a1, dn_cur[...], preferred_element_type=jnp.float32)


def _gmm(be, xs, gu, dn):
    grid_spec = pltpu.PrefetchScalarGridSpec(
        num_scalar_prefetch=1,
        grid=(NPAIR,),
        in_specs=[
            pl.BlockSpec((PBLK, D), lambda b, be: (b, 0)),
            pl.BlockSpec((E, D, 2 * F), lambda b, be: (0, 0, 0)),
            pl.BlockSpec((E, F, D), lambda b, be: (0, 0, 0)),
        ],
        out_specs=pl.BlockSpec((PBLK, D), lambda b, be: (b, 0)),
        scratch_shapes=[
            pltpu.VMEM((D, 2 * F), jnp.float32),
            pltpu.VMEM((F, D), jnp.float32),
        ],
    )
    return pl.pallas_call(
        _gmm_kernel,
        grid_spec=grid_spec,
        out_shape=jax.ShapeDtypeStruct((P, D), jnp.float32),
        compiler_params=pltpu.CompilerParams(
            dimension_semantics=("arbitrary",),
            vmem_limit_bytes=100 * 1024 * 1024,
        ),
    )(be, xs, gu, dn)


@functools.lru_cache(maxsize=None)
def _make_combine():
    mesh = plsc.VectorSubcoreMesh(core_axis_name="c", subcore_axis_name="s")

    @functools.partial(
        pl.kernel,
        mesh=mesh,
        out_type=jax.ShapeDtypeStruct((T, D), jnp.float32),
        scratch_types=[
            pltpu.VMEM((CH,), jnp.int32),
            pltpu.VMEM((CH,), jnp.int32),
            pltpu.VMEM((CH, 16), jnp.float32),
            pltpu.VMEM((SUB, D), jnp.float32),
            pltpu.VMEM((SUB, D), jnp.float32),
            pltpu.VMEM((SUB, D), jnp.float32),
            pltpu.SemaphoreType.DMA,
            pltpu.SemaphoreType.DMA,
        ],
    )
    def _combine(ys_hbm, oi_hbm, p1_hbm, p2_hbm, w1_hbm, out_hbm,
                 idx1, idx2, w1b, y1, y2, ob, sem1, sem2):
        wid = lax.axis_index("s") * 2 + lax.axis_index("c")
        base = wid * CH
        pltpu.sync_copy(p1_hbm.at[pl.ds(base, CH)], idx1)
        pltpu.sync_copy(p2_hbm.at[pl.ds(base, CH)], idx2)
        pltpu.sync_copy(w1_hbm.at[pl.ds(base, CH)], w1b)
        for sc in range(CH // SUB):
            rb = base + sc * SUB
            cp1 = pltpu.async_copy(ys_hbm.at[idx1.at[pl.ds(sc * SUB, SUB)]], y1,
                                   sem1)
            cp2 = pltpu.async_copy(ys_hbm.at[idx2.at[pl.ds(sc * SUB, SUB)]], y2,
                                   sem2)
            pltpu.sync_copy(oi_hbm.at[pl.ds(rb, SUB)], ob)
            cp1.wait()
            cp2.wait()

            def row_body(r, carry):
                tok = sc * SUB + r
                w1v = w1b[tok, :]
                for cc in range(D // 16):
                    sl = pl.ds(cc * 16, 16)
                    a = y1[r, sl]
                    b = y2[r, sl]
                    ob[r, sl] = ob[r, sl] + a + (1.0 - w1v) * (b - a)
                return carry

            lax.fori_loop(0, SUB, row_body, 0)
            pltpu.sync_copy(ob, out_hbm.at[pl.ds(rb, SUB)])

    return _combine


@jax.jit
def kernel(hidden_states, gate_w, expert_gate_up, expert_down, shared_gate_up,
           shared_down):
    p1w, p2w, w1x, bew = _route(hidden_states, gate_w)
    p1 = p1w.reshape(T)
    p2 = p2w.reshape(T)
    be = bew.reshape(NBLK)
    out_init = _shared(hidden_states, shared_gate_up, shared_down)
    xs = _make_dispatch()(hidden_states, p1, p2)
    ys = _gmm(be, xs, expert_gate_up, expert_down)
    return _make_combine()(ys, out_init, p1, p2, w1x)


# speedup vs baseline: 1.2925x; 1.0029x over previous
import functools

import jax
import jax.numpy as jnp
from jax import lax
from jax.experimental import pallas as pl
from jax.experimental.pallas import tpu as pltpu
from jax.experimental.pallas import tpu_sc as plsc

T = 2048
D = 1024
E = 8
F = 512
SF = 512

BLK = 128
PBLK = 256
P = 2 * T + E * PBLK
NBLK = P // BLK
NPAIR = P // PBLK
NW = 32
CH = T // NW
SUB = 32
CHK = 256


def _route_kernel(x_ref, gw_ref, p1_ref, p2_ref, w1_ref, be_ref):
    x = x_ref[...]
    logits = jnp.dot(x, gw_ref[...].T, preferred_element_type=jnp.float32)
    logits = logits - jnp.max(logits, axis=-1, keepdims=True)
    ex = jnp.exp(logits)
    probs = ex / jnp.sum(ex, axis=-1, keepdims=True)

    col = lax.broadcasted_iota(jnp.int32, (T, E), 1)
    a1 = jnp.argmax(probs, axis=-1)
    m1 = jnp.max(probs, axis=-1)
    oh1 = (col == a1[:, None]).astype(jnp.float32)
    masked = jnp.where(oh1 > 0, -jnp.inf, probs)
    a2 = jnp.argmax(masked, axis=-1)
    m2 = jnp.max(masked, axis=-1)
    oh2 = (col == a2[:, None]).astype(jnp.float32)
    s = m1 + m2

    M = oh1 + oh2
    ri = lax.broadcasted_iota(jnp.int32, (CHK, CHK), 0)
    ci = lax.broadcasted_iota(jnp.int32, (CHK, CHK), 1)
    tril = (ci < ri).astype(jnp.float32)
    acc = jnp.zeros((1, E), jnp.float32)
    segs = []
    for ch in range(T // CHK):
        Mc = M[ch * CHK:(ch + 1) * CHK]
        segs.append(jnp.dot(tril, Mc, preferred_element_type=jnp.float32) + acc)
        acc = acc + jnp.sum(Mc, axis=0, keepdims=True)
    S = jnp.concatenate(segs, axis=0)
    counts = acc

    padded = jnp.ceil(counts * (1.0 / PBLK)) * PBLK
    er = lax.broadcasted_iota(jnp.int32, (E, E), 0)
    ec = lax.broadcasted_iota(jnp.int32, (E, E), 1)
    upper = (er < ec).astype(jnp.float32)
    off = jnp.dot(padded, upper, preferred_element_type=jnp.float32)

    rank1 = jnp.sum(S * oh1, axis=1)
    rank2 = jnp.sum(S * oh2, axis=1)
    base1 = jnp.sum(off * oh1, axis=1)
    base2 = jnp.sum(off * oh2, axis=1)
    p1_ref[...] = (base1 + rank1).astype(jnp.int32).reshape(1, T)
    p2_ref[...] = (base2 + rank2).astype(jnp.int32).reshape(1, T)
    w1_ref[...] = jnp.broadcast_to((m1 / s)[:, None], (T, 16))

    ends = off + padded
    ends_b = jnp.broadcast_to(ends, (NBLK, E))
    sb = lax.broadcasted_iota(jnp.int32, (NBLK, E), 0).astype(
        jnp.float32) * float(BLK)
    cnt = jnp.sum((ends_b <= sb).astype(jnp.int32), axis=1)
    be_ref[...] = jnp.minimum(cnt, E - 1).reshape(1, NBLK)


def _route(x, gate_w):
    return pl.pallas_call(
        _route_kernel,
        out_shape=(
            jax.ShapeDtypeStruct((1, T), jnp.int32),
            jax.ShapeDtypeStruct((1, T), jnp.int32),
            jax.ShapeDtypeStruct((T, 16), jnp.float32),
            jax.ShapeDtypeStruct((1, NBLK), jnp.int32),
        ),
    )(x, gate_w)


def _shared_kernel(x_ref, sgu_ref, sdown_ref, out_ref):
    sh = jnp.dot(x_ref[...], sgu_ref[...], preferred_element_type=jnp.float32)
    sg = sh[:, :SF]
    su = sh[:, SF:]
    act = (sg * jax.nn.sigmoid(sg)) * su
    out_ref[...] = jnp.dot(act, sdown_ref[...], preferred_element_type=jnp.float32)


def _shared(x, sgu, sdown):
    return pl.pallas_call(
        _shared_kernel,
        out_shape=jax.ShapeDtypeStruct((T, D), jnp.float32),
    )(x, sgu, sdown)


@functools.lru_cache(maxsize=None)
def _make_dispatch():
    mesh = plsc.VectorSubcoreMesh(core_axis_name="c", subcore_axis_name="s")

    @functools.partial(
        pl.kernel,
        mesh=mesh,
        out_type=jax.ShapeDtypeStruct((P, D), jnp.float32),
        scratch_types=[
            pltpu.VMEM((CH,), jnp.int32),
            pltpu.VMEM((CH,), jnp.int32),
            pltpu.VMEM((CH, D), jnp.float32),
            pltpu.SemaphoreType.DMA,
            pltpu.SemaphoreType.DMA,
        ],
    )
    def _dispatch(x_hbm, p1_hbm, p2_hbm, xs_hbm, idx1, idx2, xbuf, sem1, sem2):
        wid = lax.axis_index("s") * 2 + lax.axis_index("c")
        base = wid * CH
        pltpu.sync_copy(p1_hbm.at[pl.ds(base, CH)], idx1)
        pltpu.sync_copy(p2_hbm.at[pl.ds(base, CH)], idx2)
        pltpu.sync_copy(x_hbm.at[pl.ds(base, CH)], xbuf)
        cp1 = pltpu.async_copy(xbuf, xs_hbm.at[idx1], sem1)
        cp2 = pltpu.async_copy(xbuf, xs_hbm.at[idx2], sem2)
        cp1.wait()
        cp2.wait()

    return _dispatch


def _gmm_kernel(be_ref, xs_ref, gu_ref, dn_ref, ys_ref):
    del be_ref
    x0 = xs_ref[:BLK]
    x1 = xs_ref[BLK:]
    h0 = jnp.dot(x0, gu_ref[0], preferred_element_type=jnp.float32)
    h1 = jnp.dot(x1, gu_ref[0], preferred_element_type=jnp.float32)
    a0 = (h0[:, :F] * jax.nn.sigmoid(h0[:, :F])) * h0[:, F:]
    a1 = (h1[:, :F] * jax.nn.sigmoid(h1[:, :F])) * h1[:, F:]
    ys_ref[:BLK] = jnp.dot(a0, dn_ref[0], preferred_element_type=jnp.float32)
    ys_ref[BLK:] = jnp.dot(a1, dn_ref[0], preferred_element_type=jnp.float32)


def _gmm(be, xs, gu, dn):
    grid_spec = pltpu.PrefetchScalarGridSpec(
        num_scalar_prefetch=1,
        grid=(NPAIR,),
        in_specs=[
            pl.BlockSpec((PBLK, D), lambda b, be: (b, 0)),
            pl.BlockSpec((1, D, 2 * F), lambda b, be: (be[2 * b], 0, 0)),
            pl.BlockSpec((1, F, D), lambda b, be: (be[2 * b], 0, 0)),
        ],
        out_specs=pl.BlockSpec((PBLK, D), lambda b, be: (b, 0)),
    )
    return pl.pallas_call(
        _gmm_kernel,
        grid_spec=grid_spec,
        out_shape=jax.ShapeDtypeStruct((P, D), jnp.float32),
        compiler_params=pltpu.CompilerParams(
            dimension_semantics=("arbitrary",),
            vmem_limit_bytes=100 * 1024 * 1024,
        ),
    )(be, xs, gu, dn)


@functools.lru_cache(maxsize=None)
def _make_combine():
    mesh = plsc.VectorSubcoreMesh(core_axis_name="c", subcore_axis_name="s")

    @functools.partial(
        pl.kernel,
        mesh=mesh,
        out_type=jax.ShapeDtypeStruct((T, D), jnp.float32),
        scratch_types=[
            pltpu.VMEM((CH,), jnp.int32),
            pltpu.VMEM((CH,), jnp.int32),
            pltpu.VMEM((CH, 16), jnp.float32),
            pltpu.VMEM((SUB, D), jnp.float32),
            pltpu.VMEM((SUB, D), jnp.float32),
            pltpu.VMEM((SUB, D), jnp.float32),
            pltpu.SemaphoreType.DMA,
            pltpu.SemaphoreType.DMA,
        ],
    )
    def _combine(ys_hbm, oi_hbm, p1_hbm, p2_hbm, w1_hbm, out_hbm,
                 idx1, idx2, w1b, y1, y2, ob, sem1, sem2):
        wid = lax.axis_index("s") * 2 + lax.axis_index("c")
        base = wid * CH
        pltpu.sync_copy(p1_hbm.at[pl.ds(base, CH)], idx1)
        pltpu.sync_copy(p2_hbm.at[pl.ds(base, CH)], idx2)
        pltpu.sync_copy(w1_hbm.at[pl.ds(base, CH)], w1b)
        for sc in range(CH // SUB):
            rb = base + sc * SUB
            cp1 = pltpu.async_copy(ys_hbm.at[idx1.at[pl.ds(sc * SUB, SUB)]], y1,
                                   sem1)
            cp2 = pltpu.async_copy(ys_hbm.at[idx2.at[pl.ds(sc * SUB, SUB)]], y2,
                                   sem2)
            pltpu.sync_copy(oi_hbm.at[pl.ds(rb, SUB)], ob)
            cp1.wait()
            cp2.wait()

            def row_body(r, carry):
                tok = sc * SUB + r
                w1v = w1b[tok, :]
                for cc in range(D // 16):
                    sl = pl.ds(cc * 16, 16)
                    a = y1[r, sl]
                    b = y2[r, sl]
                    ob[r, sl] = ob[r, sl] + a + (1.0 - w1v) * (b - a)
                return carry

            lax.fori_loop(0, SUB, row_body, 0)
            pltpu.sync_copy(ob, out_hbm.at[pl.ds(rb, SUB)])

    return _combine


@jax.jit
def kernel(hidden_states, gate_w, expert_gate_up, expert_down, shared_gate_up,
           shared_down):
    p1w, p2w, w1x, bew = _route(hidden_states, gate_w)
    p1 = p1w.reshape(T)
    p2 = p2w.reshape(T)
    be = bew.reshape(NBLK)
    out_init = _shared(hidden_states, shared_gate_up, shared_down)
    xs = _make_dispatch()(hidden_states, p1, p2)
    ys = _gmm(be, xs, expert_gate_up, expert_down)
    return _make_combine()(ys, out_init, p1, p2, w1x)


# speedup vs baseline: 1.3503x; 1.0447x over previous
import functools

import jax
import jax.numpy as jnp
from jax import lax
from jax.experimental import pallas as pl
from jax.experimental.pallas import tpu as pltpu
from jax.experimental.pallas import tpu_sc as plsc

T = 2048
D = 1024
E = 8
F = 512
SF = 512

BLK = 128
PBLK = 512
P = 2 * T + E * PBLK
NBLK = P // BLK
NPAIR = P // PBLK
NW = 32
CH = T // NW
SUB = 32
CHK = 256


def _route_kernel(x_ref, gw_ref, p1_ref, p2_ref, w1_ref, be_ref):
    x = x_ref[...]
    logits = jnp.dot(x, gw_ref[...].T, preferred_element_type=jnp.float32)
    logits = logits - jnp.max(logits, axis=-1, keepdims=True)
    ex = jnp.exp(logits)
    probs = ex / jnp.sum(ex, axis=-1, keepdims=True)

    col = lax.broadcasted_iota(jnp.int32, (T, E), 1)
    a1 = jnp.argmax(probs, axis=-1)
    m1 = jnp.max(probs, axis=-1)
    oh1 = (col == a1[:, None]).astype(jnp.float32)
    masked = jnp.where(oh1 > 0, -jnp.inf, probs)
    a2 = jnp.argmax(masked, axis=-1)
    m2 = jnp.max(masked, axis=-1)
    oh2 = (col == a2[:, None]).astype(jnp.float32)
    s = m1 + m2

    M = oh1 + oh2
    ri = lax.broadcasted_iota(jnp.int32, (CHK, CHK), 0)
    ci = lax.broadcasted_iota(jnp.int32, (CHK, CHK), 1)
    tril = (ci < ri).astype(jnp.float32)
    acc = jnp.zeros((1, E), jnp.float32)
    segs = []
    for ch in range(T // CHK):
        Mc = M[ch * CHK:(ch + 1) * CHK]
        segs.append(jnp.dot(tril, Mc, preferred_element_type=jnp.float32) + acc)
        acc = acc + jnp.sum(Mc, axis=0, keepdims=True)
    S = jnp.concatenate(segs, axis=0)
    counts = acc

    padded = jnp.ceil(counts * (1.0 / PBLK)) * PBLK
    er = lax.broadcasted_iota(jnp.int32, (E, E), 0)
    ec = lax.broadcasted_iota(jnp.int32, (E, E), 1)
    upper = (er < ec).astype(jnp.float32)
    off = jnp.dot(padded, upper, preferred_element_type=jnp.float32)

    rank1 = jnp.sum(S * oh1, axis=1)
    rank2 = jnp.sum(S * oh2, axis=1)
    base1 = jnp.sum(off * oh1, axis=1)
    base2 = jnp.sum(off * oh2, axis=1)
    p1_ref[...] = (base1 + rank1).astype(jnp.int32).reshape(1, T)
    p2_ref[...] = (base2 + rank2).astype(jnp.int32).reshape(1, T)
    w1_ref[...] = jnp.broadcast_to((m1 / s)[:, None], (T, 16))

    ends = off + padded
    ends_b = jnp.broadcast_to(ends, (NBLK, E))
    sb = lax.broadcasted_iota(jnp.int32, (NBLK, E), 0).astype(
        jnp.float32) * float(BLK)
    cnt = jnp.sum((ends_b <= sb).astype(jnp.int32), axis=1)
    be_ref[...] = jnp.minimum(cnt, E - 1).reshape(1, NBLK)


def _route(x, gate_w):
    return pl.pallas_call(
        _route_kernel,
        out_shape=(
            jax.ShapeDtypeStruct((1, T), jnp.int32),
            jax.ShapeDtypeStruct((1, T), jnp.int32),
            jax.ShapeDtypeStruct((T, 16), jnp.float32),
            jax.ShapeDtypeStruct((1, NBLK), jnp.int32),
        ),
    )(x, gate_w)


def _shared_kernel(x_ref, sgu_ref, sdown_ref, out_ref):
    sh = jnp.dot(x_ref[...], sgu_ref[...], preferred_element_type=jnp.float32)
    sg = sh[:, :SF]
    su = sh[:, SF:]
    act = (sg * jax.nn.sigmoid(sg)) * su
    out_ref[...] = jnp.dot(act, sdown_ref[...], preferred_element_type=jnp.float32)


def _shared(x, sgu, sdown):
    return pl.pallas_call(
        _shared_kernel,
        out_shape=jax.ShapeDtypeStruct((T, D), jnp.float32),
    )(x, sgu, sdown)


@functools.lru_cache(maxsize=None)
def _make_dispatch():
    mesh = plsc.VectorSubcoreMesh(core_axis_name="c", subcore_axis_name="s")

    @functools.partial(
        pl.kernel,
        mesh=mesh,
        out_type=jax.ShapeDtypeStruct((P, D), jnp.float32),
        scratch_types=[
            pltpu.VMEM((CH,), jnp.int32),
            pltpu.VMEM((CH,), jnp.int32),
            pltpu.VMEM((CH, D), jnp.float32),
            pltpu.SemaphoreType.DMA,
            pltpu.SemaphoreType.DMA,
        ],
    )
    def _dispatch(x_hbm, p1_hbm, p2_hbm, xs_hbm, idx1, idx2, xbuf, sem1, sem2):
        wid = lax.axis_index("s") * 2 + lax.axis_index("c")
        base = wid * CH
        pltpu.sync_copy(p1_hbm.at[pl.ds(base, CH)], idx1)
        pltpu.sync_copy(p2_hbm.at[pl.ds(base, CH)], idx2)
        pltpu.sync_copy(x_hbm.at[pl.ds(base, CH)], xbuf)
        cp1 = pltpu.async_copy(xbuf, xs_hbm.at[idx1], sem1)
        cp2 = pltpu.async_copy(xbuf, xs_hbm.at[idx2], sem2)
        cp1.wait()
        cp2.wait()

    return _dispatch


def _gmm_kernel(be_ref, xs_ref, gu_ref, dn_ref, ys_ref):
    del be_ref
    HB = PBLK // 2
    x0 = xs_ref[:HB]
    x1 = xs_ref[HB:]
    h0 = jnp.dot(x0, gu_ref[0], preferred_element_type=jnp.float32)
    h1 = jnp.dot(x1, gu_ref[0], preferred_element_type=jnp.float32)
    a0 = (h0[:, :F] * jax.nn.sigmoid(h0[:, :F])) * h0[:, F:]
    a1 = (h1[:, :F] * jax.nn.sigmoid(h1[:, :F])) * h1[:, F:]
    ys_ref[:HB] = jnp.dot(a0, dn_ref[0], preferred_element_type=jnp.float32)
    ys_ref[HB:] = jnp.dot(a1, dn_ref[0], preferred_element_type=jnp.float32)


def _gmm(be, xs, gu, dn):
    grid_spec = pltpu.PrefetchScalarGridSpec(
        num_scalar_prefetch=1,
        grid=(NPAIR,),
        in_specs=[
            pl.BlockSpec((PBLK, D), lambda b, be: (b, 0)),
            pl.BlockSpec((1, D, 2 * F),
                         lambda b, be: (be[(PBLK // BLK) * b], 0, 0)),
            pl.BlockSpec((1, F, D),
                         lambda b, be: (be[(PBLK // BLK) * b], 0, 0)),
        ],
        out_specs=pl.BlockSpec((PBLK, D), lambda b, be: (b, 0)),
    )
    return pl.pallas_call(
        _gmm_kernel,
        grid_spec=grid_spec,
        out_shape=jax.ShapeDtypeStruct((P, D), jnp.float32),
        compiler_params=pltpu.CompilerParams(
            dimension_semantics=("arbitrary",),
            vmem_limit_bytes=100 * 1024 * 1024,
        ),
    )(be, xs, gu, dn)


@functools.lru_cache(maxsize=None)
def _make_combine():
    mesh = plsc.VectorSubcoreMesh(core_axis_name="c", subcore_axis_name="s")

    @functools.partial(
        pl.kernel,
        mesh=mesh,
        out_type=jax.ShapeDtypeStruct((T, D), jnp.float32),
        scratch_types=[
            pltpu.VMEM((CH,), jnp.int32),
            pltpu.VMEM((CH,), jnp.int32),
            pltpu.VMEM((CH, 16), jnp.float32),
            pltpu.VMEM((SUB, D), jnp.float32),
            pltpu.VMEM((SUB, D), jnp.float32),
            pltpu.VMEM((SUB, D), jnp.float32),
            pltpu.SemaphoreType.DMA,
            pltpu.SemaphoreType.DMA,
        ],
    )
    def _combine(ys_hbm, oi_hbm, p1_hbm, p2_hbm, w1_hbm, out_hbm,
                 idx1, idx2, w1b, y1, y2, ob, sem1, sem2):
        wid = lax.axis_index("s") * 2 + lax.axis_index("c")
        base = wid * CH
        pltpu.sync_copy(p1_hbm.at[pl.ds(base, CH)], idx1)
        pltpu.sync_copy(p2_hbm.at[pl.ds(base, CH)], idx2)
        pltpu.sync_copy(w1_hbm.at[pl.ds(base, CH)], w1b)
        for sc in range(CH // SUB):
            rb = base + sc * SUB
            cp1 = pltpu.async_copy(ys_hbm.at[idx1.at[pl.ds(sc * SUB, SUB)]], y1,
                                   sem1)
            cp2 = pltpu.async_copy(ys_hbm.at[idx2.at[pl.ds(sc * SUB, SUB)]], y2,
                                   sem2)
            pltpu.sync_copy(oi_hbm.at[pl.ds(rb, SUB)], ob)
            cp1.wait()
            cp2.wait()

            def row_body(r, carry):
                tok = sc * SUB + r
                w1v = w1b[tok, :]
                for cc in range(D // 16):
                    sl = pl.ds(cc * 16, 16)
                    a = y1[r, sl]
                    b = y2[r, sl]
                    ob[r, sl] = ob[r, sl] + a + (1.0 - w1v) * (b - a)
                return carry

            lax.fori_loop(0, SUB, row_body, 0)
            pltpu.sync_copy(ob, out_hbm.at[pl.ds(rb, SUB)])

    return _combine


@jax.jit
def kernel(hidden_states, gate_w, expert_gate_up, expert_down, shared_gate_up,
           shared_down):
    p1w, p2w, w1x, bew = _route(hidden_states, gate_w)
    p1 = p1w.reshape(T)
    p2 = p2w.reshape(T)
    be = bew.reshape(NBLK)
    out_init = _shared(hidden_states, shared_gate_up, shared_down)
    xs = _make_dispatch()(hidden_states, p1, p2)
    ys = _gmm(be, xs, expert_gate_up, expert_down)
    return _make_combine()(ys, out_init, p1, p2, w1x)
